# Initial kernel scaffold; baseline (speedup 1.0000x reference)
import jax
import jax.numpy as jnp
from jax.experimental import pallas as pl


def kernel(x, w_emb, w_feat, w_main, up, wc, scale):
    raise NotImplementedError("write your pallas kernel here")


# speedup vs baseline: 1.8010x; 1.8010x over previous
import jax
import jax.numpy as jnp
from jax.experimental import pallas as pl
from jax.experimental.pallas import tpu as pltpu

_FEATURE_DILATIONS = (1, 3, 9, 1, 1, 1)
_MAIN_DILATIONS = (1, 3, 9, 1, 3, 9, 1, 1)
_KSIZE = 3
_NEG_SLOPE = 0.2


def _leaky(v):
    return jnp.where(v >= 0.0, v, _NEG_SLOPE * v)


def _make_body(channels, fb_taps, l_in, l_out, lead):
    C = channels

    def dilated_layer(e, w, d, slab, L):
        pad = _KSIZE * d // 2
        eb = e.astype(jnp.bfloat16)
        for j in range(_KSIZE):
            s = pad - j * d
            b = j * C
            if s > 0:
                slab[b:b + C, 0:s] = jnp.zeros((C, s), jnp.bfloat16)
                slab[b:b + C, s:L] = eb[:, 0:L - s]
            elif s < 0:
                slab[b:b + C, 0:L + s] = eb[:, -s:L]
                slab[b:b + C, L + s:L] = jnp.zeros((C, -s), jnp.bfloat16)
            else:
                slab[b:b + C, 0:L] = eb
        t = jnp.dot(w, slab[:, 0:L], preferred_element_type=jnp.float32)
        return _leaky(e + t)

    def body(x_ref, w_emb_ref, w_feat_ref, w_main_ref, up_ref, wc_ref,
             scale_ref, o_ref, fslab, mslab):
        x = x_ref[0].astype(jnp.bfloat16)
        e = _leaky(jnp.dot(w_emb_ref[...], x,
                           preferred_element_type=jnp.float32))
        for li, d in enumerate(_FEATURE_DILATIONS):
            e = dilated_layer(e, w_feat_ref[li], d, fslab, l_in)
        e = jnp.dot(e.astype(jnp.bfloat16), up_ref[...],
                    preferred_element_type=jnp.float32)
        for li, d in enumerate(_MAIN_DILATIONS):
            e = dilated_layer(e, w_main_ref[li], d, mslab, l_out)
        g = jnp.dot(wc_ref[...], e.astype(jnp.bfloat16),
                    preferred_element_type=jnp.float32)
        acc = None
        for k in range(fb_taps):
            s = lead - k
            row = g[k:k + 1, :]
            if s > 0:
                piece = jnp.concatenate(
                    [jnp.zeros((1, s), jnp.float32), row[:, 0:l_out - s]],
                    axis=1)
            elif s < 0:
                piece = jnp.concatenate(
                    [row[:, -s:l_out], jnp.zeros((1, -s), jnp.float32)],
                    axis=1)
            else:
                piece = row
            acc = piece if acc is None else acc + piece
        o_ref[0] = acc * jnp.abs(scale_ref[0])

    return body


def _build_forward(batch, channels, c_sl, fb_taps, l_in, l_out, lead,
                   n_feat, n_main):
    body = _make_body(channels, fb_taps, l_in, l_out, lead)
    grid_spec = pltpu.PrefetchScalarGridSpec(
        num_scalar_prefetch=0,
        grid=(batch,),
        in_specs=[
            pl.BlockSpec((1, c_sl, l_in), lambda b: (b, 0, 0)),
            pl.BlockSpec((channels, c_sl), lambda b: (0, 0)),
            pl.BlockSpec((n_feat, channels, _KSIZE * channels),
                         lambda b: (0, 0, 0)),
            pl.BlockSpec((n_main, channels, _KSIZE * channels),
                         lambda b: (0, 0, 0)),
            pl.BlockSpec((l_in, l_out), lambda b: (0, 0)),
            pl.BlockSpec((fb_taps, channels), lambda b: (0, 0)),
            pl.BlockSpec(memory_space=pltpu.MemorySpace.SMEM),
        ],
        out_specs=pl.BlockSpec((1, 1, l_out), lambda b: (b, 0, 0)),
        scratch_shapes=[
            pltpu.VMEM((_KSIZE * channels, l_in), jnp.bfloat16),
            pltpu.VMEM((_KSIZE * channels, l_out), jnp.bfloat16),
        ],
    )
    return pl.pallas_call(
        body,
        grid_spec=grid_spec,
        out_shape=jax.ShapeDtypeStruct((batch, 1, l_out), jnp.float32),
        compiler_params=pltpu.CompilerParams(
            dimension_semantics=("parallel",),
            vmem_limit_bytes=48 * 2**20),
    )


def kernel(x, w_emb, w_feat, w_main, up, wc, scale):
    batch = x.shape[0]
    channels, c_sl = w_emb.shape
    l_in, l_out = up.shape
    fb_taps = wc.shape[0]
    lead = fb_taps - fb_taps // 2
    n_feat = w_feat.shape[0]
    n_main = w_main.shape[0]

    xs = x.reshape(batch, -1, l_in)[:, 0:c_sl, :].astype(jnp.float32)
    fwd = _build_forward(batch, channels, c_sl, fb_taps, l_in, l_out, lead,
                         n_feat, n_main)
    out = fwd(xs,
              w_emb.astype(jnp.bfloat16),
              w_feat.astype(jnp.bfloat16),
              w_main.astype(jnp.bfloat16),
              up.astype(jnp.bfloat16),
              wc.astype(jnp.bfloat16),
              scale)
    return out
